# Initial kernel scaffold; baseline (speedup 1.0000x reference)
import jax
import jax.numpy as jnp
from jax.experimental import pallas as pl
from jax.experimental.pallas import tpu as pltpu

NUM_PROMPTS = 64
NUM_DIMS = 4096
VOCAB = 100000
EPS = 1e-8
BV = 1000


def _knn_kernel(q_ref, k_ref, out_ref, best_val, best_idx):
    i = pl.program_id(0)
    nsteps = pl.num_programs(0)

    kblk = k_ref[...]
    scores = jax.lax.dot_general(
        q_ref[...], kblk,
        dimension_numbers=(((1,), (1,)), ((), ())),
        preferred_element_type=jnp.float32,
        precision=jax.lax.Precision.HIGHEST,
    )
    sumsq = jnp.sum(kblk * kblk, axis=1)
    inv = 1.0 / jnp.maximum(jnp.sqrt(sumsq), EPS)
    scores = scores * inv[None, :]

    m = jnp.max(scores, axis=1, keepdims=True)
    a = jnp.argmax(scores, axis=1).astype(jnp.int32)
    a = a[:, None] + i * BV

    @pl.when(i == 0)
    def _init():
        best_val[...] = m
        best_idx[...] = a

    @pl.when(i != 0)
    def _merge():
        prev = best_val[...]
        take = m > prev
        best_val[...] = jnp.where(take, m, prev)
        best_idx[...] = jnp.where(take, a, best_idx[...])

    @pl.when(i == nsteps - 1)
    def _finish():
        out_ref[...] = best_idx[...]


@jax.jit
def kernel(embeddings, embedding_weight):
    out = pl.pallas_call(
        _knn_kernel,
        grid=(VOCAB // BV,),
        in_specs=[
            pl.BlockSpec((NUM_PROMPTS, NUM_DIMS), lambda i: (0, 0)),
            pl.BlockSpec((BV, NUM_DIMS), lambda i: (i, 0)),
        ],
        out_specs=pl.BlockSpec((NUM_PROMPTS, 1), lambda i: (0, 0)),
        out_shape=jax.ShapeDtypeStruct((NUM_PROMPTS, 1), jnp.int32),
        scratch_shapes=[
            pltpu.VMEM((NUM_PROMPTS, 1), jnp.float32),
            pltpu.VMEM((NUM_PROMPTS, 1), jnp.int32),
        ],
    )(embeddings, embedding_weight)
    return out[:, 0]


# speedup vs baseline: 1.9747x; 1.9747x over previous
import jax
import jax.numpy as jnp
from jax.experimental import pallas as pl
from jax.experimental.pallas import tpu as pltpu

NUM_PROMPTS = 64
NUM_DIMS = 4096
VOCAB = 100000
EPS = 1e-8
BV = 1000


def _knn_kernel(q_ref, k_ref, out_ref, qn_ref, best_val, best_idx):
    i = pl.program_id(0)
    nsteps = pl.num_programs(0)

    @pl.when(i == 0)
    def _norm_q():
        q = q_ref[...]
        qn = jnp.maximum(jnp.sqrt(jnp.sum(q * q, axis=1, keepdims=True)), EPS)
        qn_ref[...] = q / qn

    kblk = k_ref[...]
    knorm = jnp.maximum(jnp.sqrt(jnp.sum(kblk * kblk, axis=1, keepdims=True)), EPS)
    kn = kblk / knorm
    scores = jax.lax.dot_general(
        qn_ref[...], kn,
        dimension_numbers=(((1,), (1,)), ((), ())),
        preferred_element_type=jnp.float32,
    )

    m = jnp.max(scores, axis=1, keepdims=True)
    a = jnp.argmax(scores, axis=1).astype(jnp.int32)
    a = a[:, None] + i * BV

    @pl.when(i == 0)
    def _init():
        best_val[...] = m
        best_idx[...] = a

    @pl.when(i != 0)
    def _merge():
        prev = best_val[...]
        take = m > prev
        best_val[...] = jnp.where(take, m, prev)
        best_idx[...] = jnp.where(take, a, best_idx[...])

    @pl.when(i == nsteps - 1)
    def _finish():
        out_ref[...] = best_idx[...]


@jax.jit
def kernel(embeddings, embedding_weight):
    out = pl.pallas_call(
        _knn_kernel,
        grid=(VOCAB // BV,),
        in_specs=[
            pl.BlockSpec((NUM_PROMPTS, NUM_DIMS), lambda i: (0, 0)),
            pl.BlockSpec((BV, NUM_DIMS), lambda i: (i, 0)),
        ],
        out_specs=pl.BlockSpec((NUM_PROMPTS, 1), lambda i: (0, 0)),
        out_shape=jax.ShapeDtypeStruct((NUM_PROMPTS, 1), jnp.int32),
        scratch_shapes=[
            pltpu.VMEM((NUM_PROMPTS, NUM_DIMS), jnp.float32),
            pltpu.VMEM((NUM_PROMPTS, 1), jnp.float32),
            pltpu.VMEM((NUM_PROMPTS, 1), jnp.int32),
        ],
    )(embeddings, embedding_weight)
    return out[:, 0]
